# Initial kernel scaffold; baseline (speedup 1.0000x reference)
import functools

import jax
import jax.numpy as jnp
from jax import lax
from jax.experimental import pallas as pl
from jax.experimental.pallas import tpu as pltpu
from jax.experimental.pallas import tpu_sc as plsc

_N = 10000
_E = 320000
_H = 128
_G = 64
_NT = 10

_NC = 2
_NS = 16
_NW = _NC * _NS
_EPW = _E // _NW
_CH = 80
_NCHUNK = _EPW // _CH
_NPAD = 10240
_RPT = _NPAD // _NS
_ZR = 128
_DW = 16
_BN = 1000


_sc_mesh = plsc.VectorSubcoreMesh(core_axis_name="c", subcore_axis_name="s")


@functools.partial(
    pl.kernel,
    mesh=_sc_mesh,
    out_type=jax.ShapeDtypeStruct((_NC, _NPAD, _DW), jnp.float32),
    scratch_types=[
        pltpu.VMEM((_CH,), jnp.int32),
        pltpu.VMEM((_CH, _DW), jnp.float32),
        pltpu.VMEM((_RPT, _DW), jnp.float32),
        pltpu.VMEM_SHARED((_NPAD, _DW), jnp.float32),
    ],
)
def _sc_deg(dst_hbm, ones_hbm, zd_hbm, out_hbm, didx, ones_v, zd_v, acc):
    c = lax.axis_index("c")
    s = lax.axis_index("s")
    wid = c * _NS + s
    pltpu.sync_copy(zd_hbm, zd_v)
    pltpu.sync_copy(zd_v, acc.at[pl.ds(s * _RPT, _RPT), :])
    pltpu.sync_copy(ones_hbm, ones_v)
    plsc.subcore_barrier()
    base = wid * _EPW

    def body(j, carry):
        pltpu.sync_copy(dst_hbm.at[pl.ds(base + j * _CH, _CH)], didx)
        pltpu.sync_copy(ones_v, acc.at[didx], add=True)
        return carry

    lax.fori_loop(0, _NCHUNK, body, 0)
    plsc.subcore_barrier()
    pltpu.sync_copy(acc.at[pl.ds(s * _RPT, _RPT), :], zd_v)
    pltpu.sync_copy(zd_v, out_hbm.at[c, pl.ds(s * _RPT, _RPT), :])


@functools.partial(
    pl.kernel,
    mesh=_sc_mesh,
    out_type=jax.ShapeDtypeStruct((_NC, _NPAD, _H), jnp.float32),
    scratch_types=[
        pltpu.VMEM((_CH,), jnp.int32),
        pltpu.VMEM((_CH,), jnp.int32),
        pltpu.VMEM((_CH, _H), jnp.float32),
        pltpu.VMEM((_ZR, _H), jnp.float32),
        pltpu.VMEM_SHARED((_NPAD, _H), jnp.float32),
        pltpu.SemaphoreType.DMA,
    ],
)
def _sc_edge(g_hbm, src_hbm, dst_hbm, z_hbm, out_hbm, sidx, didx, rows, zrows,
             acc, sem):
    c = lax.axis_index("c")
    s = lax.axis_index("s")
    wid = c * _NS + s
    pltpu.sync_copy(z_hbm, zrows)
    for k in range(_RPT // _ZR):
        pltpu.sync_copy(zrows, acc.at[pl.ds(s * _RPT + k * _ZR, _ZR), :])
    plsc.subcore_barrier()
    base = wid * _EPW

    def body(j, carry):
        e0 = base + j * _CH
        pltpu.sync_copy(src_hbm.at[pl.ds(e0, _CH)], sidx)
        pltpu.async_copy(g_hbm.at[sidx], rows, sem).wait()
        pltpu.sync_copy(dst_hbm.at[pl.ds(e0, _CH)], didx)
        pltpu.sync_copy(rows, acc.at[didx], add=True)
        return carry

    lax.fori_loop(0, _NCHUNK, body, 0)
    plsc.subcore_barrier()
    for k in range(_RPT // _ZR):
        r0 = s * _RPT + k * _ZR
        pltpu.sync_copy(acc.at[pl.ds(r0, _ZR), :], zrows)
        pltpu.sync_copy(zrows, out_hbm.at[c, pl.ds(r0, _ZR), :])


def _tc1_body(x_r, te_r, d2_r, w1_r, wt_r, bt_r, g1_o, te_o, dv_o):
    d2 = d2_r[...]
    deg = (jnp.sum(d2[0], axis=1) + jnp.sum(d2[1], axis=1)) * (1.0 / _DW) + 1.0
    dinv = lax.rsqrt(deg)[:, None]
    g1_o[...] = jnp.dot(x_r[...], w1_r[...],
                        preferred_element_type=jnp.float32) * dinv
    te_o[...] = jnp.maximum(
        jnp.dot(te_r[...], wt_r[...], preferred_element_type=jnp.float32)
        + bt_r[...], 0.0)
    dv_o[...] = dinv


def _tc_mid_temb_body(s_r, g_r, dv_r, b_r, w_r, te_r, gn_o):
    sr = s_r[...]
    dv = dv_r[...]
    h = jnp.maximum((sr[0] + sr[1] + g_r[...]) * dv + b_r[...], 0.0) + te_r[...]
    gn_o[...] = jnp.dot(h, w_r[...], preferred_element_type=jnp.float32) * dv


def _tc_mid_body(s_r, g_r, dv_r, b_r, w_r, gn_o):
    sr = s_r[...]
    dv = dv_r[...]
    h = jnp.maximum((sr[0] + sr[1] + g_r[...]) * dv + b_r[...], 0.0)
    gn_o[...] = jnp.dot(h, w_r[...], preferred_element_type=jnp.float32) * dv


def _tc_pool_body(s_r, g_r, dv_r, b_r, ba_r, ms_o, mc_o):
    i = pl.program_id(0)
    sr = s_r[...]
    h = jnp.maximum((sr[0] + sr[1] + g_r[...]) * dv_r[...] + b_r[...], 0.0)
    bb = ba_r[0]
    gids = lax.broadcasted_iota(jnp.int32, (_G, _BN), 0)
    mask = (gids == bb).astype(jnp.float32)
    ps = jnp.dot(mask, h, preferred_element_type=jnp.float32)
    pc = jnp.broadcast_to(jnp.sum(mask, axis=1, keepdims=True), (_G, _H))

    @pl.when(i == 0)
    def _():
        ms_o[...] = ps
        mc_o[...] = pc

    @pl.when(i != 0)
    def _():
        ms_o[...] = ms_o[...] + ps
        mc_o[...] = mc_o[...] + pc


def _tc_out_body(ms_r, mc_r, wo_r, bo_r, o_r):
    pooled = ms_r[...] / jnp.maximum(mc_r[...], 1.0)
    o_r[...] = jnp.dot(pooled, wo_r[...],
                       preferred_element_type=jnp.float32) + bo_r[...]


def _row_spec(i):
    return (i, 0)


def kernel(x, edge_index, t_embedding, batch, Wt, bt, W1, b1, W2, b2, W3, b3,
           Wo, bo):
    src = edge_index[0]
    dst = edge_index[1]
    zrows = jnp.zeros((_ZR, _H), jnp.float32)
    zd = jnp.zeros((_RPT, _DW), jnp.float32)
    onesd = jnp.ones((_CH, _DW), jnp.float32)

    deg2 = _sc_deg(dst, onesd, zd)

    grid = (_N // _BN,)
    row = pl.BlockSpec((_BN, _H), _row_spec)
    col1 = pl.BlockSpec((_BN, 1), _row_spec)
    wsp = pl.BlockSpec((_H, _H), lambda i: (0, 0))
    bsp = pl.BlockSpec((1, _H), lambda i: (0, 0))
    ssp = pl.BlockSpec((_NC, _BN, _H), lambda i: (0, i, 0))

    g1, temb, dinv = pl.pallas_call(
        _tc1_body,
        grid=grid,
        in_specs=[row, row,
                  pl.BlockSpec((_NC, _BN, _DW), lambda i: (0, i, 0)),
                  wsp, wsp, bsp],
        out_specs=[row, row, col1],
        out_shape=[jax.ShapeDtypeStruct((_N, _H), jnp.float32),
                   jax.ShapeDtypeStruct((_N, _H), jnp.float32),
                   jax.ShapeDtypeStruct((_N, 1), jnp.float32)],
    )(x, t_embedding, deg2, W1, Wt, bt.reshape(1, _H))

    s1 = _sc_edge(g1, src, dst, zrows)

    g2 = pl.pallas_call(
        _tc_mid_temb_body,
        grid=grid,
        in_specs=[ssp, row, col1, bsp, wsp, row],
        out_specs=row,
        out_shape=jax.ShapeDtypeStruct((_N, _H), jnp.float32),
    )(s1, g1, dinv, b1.reshape(1, _H), W2, temb)

    s2 = _sc_edge(g2, src, dst, zrows)

    g3 = pl.pallas_call(
        _tc_mid_body,
        grid=grid,
        in_specs=[ssp, row, col1, bsp, wsp],
        out_specs=row,
        out_shape=jax.ShapeDtypeStruct((_N, _H), jnp.float32),
    )(s2, g2, dinv, b2.reshape(1, _H), W3)

    s3 = _sc_edge(g3, src, dst, zrows)

    msum, mcnt = pl.pallas_call(
        _tc_pool_body,
        grid=grid,
        in_specs=[ssp, row, col1, bsp,
                  pl.BlockSpec((1, 1, _BN), lambda i: (i, 0, 0))],
        out_specs=[pl.BlockSpec((_G, _H), lambda i: (0, 0)),
                   pl.BlockSpec((_G, _H), lambda i: (0, 0))],
        out_shape=[jax.ShapeDtypeStruct((_G, _H), jnp.float32),
                   jax.ShapeDtypeStruct((_G, _H), jnp.float32)],
    )(s3, g3, dinv, b3.reshape(1, _H),
      batch.reshape(_N // _BN, 1, _BN))

    wo_pad = jnp.zeros((_H, _H), jnp.float32).at[:, :_NT].set(Wo)
    bo_pad = jnp.zeros((1, _H), jnp.float32).at[0, :_NT].set(bo)

    out = pl.pallas_call(
        _tc_out_body,
        grid=(1,),
        in_specs=[pl.BlockSpec((_G, _H), lambda i: (0, 0)),
                  pl.BlockSpec((_G, _H), lambda i: (0, 0)),
                  wsp, bsp],
        out_specs=pl.BlockSpec((_G, _H), lambda i: (0, 0)),
        out_shape=jax.ShapeDtypeStruct((_G, _H), jnp.float32),
    )(msum, mcnt, wo_pad, bo_pad)

    return out[:, :_NT]


# speedup vs baseline: 10.6800x; 10.6800x over previous
import functools

import jax
import jax.numpy as jnp
from jax import lax
from jax.experimental import pallas as pl
from jax.experimental.pallas import tpu as pltpu
from jax.experimental.pallas import tpu_sc as plsc

_N = 10000
_E = 320000
_H = 128
_G = 64
_NT = 10

_NC = 2
_NS = 16
_NW = _NC * _NS
_EPW = _E // _NW
_CH = 80
_NCHUNK = _EPW // _CH
_NPAD = 10240
_BN = 1000


_sc_mesh = plsc.VectorSubcoreMesh(core_axis_name="c", subcore_axis_name="s")


@functools.partial(
    pl.kernel,
    mesh=_sc_mesh,
    out_type=jax.ShapeDtypeStruct((_NC, _NPAD, _H), jnp.float32),
    scratch_types=[
        pltpu.VMEM((_CH,), jnp.int32),
        pltpu.VMEM((_CH, _H), jnp.float32),
        pltpu.VMEM_SHARED((_NPAD, _H), jnp.float32),
    ],
)
def _sc_deg(dst_hbm, ones_hbm, z_hbm, out_hbm, didx, ones_v, acc):
    c = lax.axis_index("c")
    s = lax.axis_index("s")
    wid = c * _NS + s

    @pl.when(s == 0)
    def _():
        pltpu.sync_copy(z_hbm, acc)

    pltpu.sync_copy(ones_hbm, ones_v)
    plsc.subcore_barrier()
    base = wid * _EPW

    def body(j, carry):
        pltpu.sync_copy(dst_hbm.at[pl.ds(base + j * _CH, _CH)], didx)
        pltpu.sync_copy(ones_v, acc.at[didx], add=True)
        return carry

    lax.fori_loop(0, _NCHUNK, body, 0)
    plsc.subcore_barrier()

    @pl.when(s == 0)
    def _():
        pltpu.sync_copy(acc, out_hbm.at[c])


@functools.partial(
    pl.kernel,
    mesh=_sc_mesh,
    out_type=jax.ShapeDtypeStruct((_NC, _NPAD, _H), jnp.float32),
    scratch_types=[
        pltpu.VMEM((_CH,), jnp.int32),
        pltpu.VMEM((_CH,), jnp.int32),
        pltpu.VMEM((_CH, _H), jnp.float32),
        pltpu.VMEM_SHARED((_NPAD, _H), jnp.float32),
        pltpu.SemaphoreType.DMA,
    ],
)
def _sc_edge(g_hbm, src_hbm, dst_hbm, z_hbm, out_hbm, sidx, didx, rows, acc,
             sem):
    c = lax.axis_index("c")
    s = lax.axis_index("s")
    wid = c * _NS + s

    @pl.when(s == 0)
    def _():
        pltpu.sync_copy(z_hbm, acc)

    plsc.subcore_barrier()
    base = wid * _EPW

    def body(j, carry):
        e0 = base + j * _CH
        pltpu.sync_copy(src_hbm.at[pl.ds(e0, _CH)], sidx)
        pltpu.async_copy(g_hbm.at[sidx], rows, sem).wait()
        pltpu.sync_copy(dst_hbm.at[pl.ds(e0, _CH)], didx)
        pltpu.sync_copy(rows, acc.at[didx], add=True)
        return carry

    lax.fori_loop(0, _NCHUNK, body, 0)
    plsc.subcore_barrier()

    @pl.when(s == 0)
    def _():
        pltpu.sync_copy(acc, out_hbm.at[c])


def _tc1_body(x_r, te_r, d2_r, w1_r, wt_r, bt_r, g1_o, te_o, dv_o):
    d2 = d2_r[...]
    deg = jnp.sum(d2[0] + d2[1], axis=1) * (1.0 / _H) + 1.0
    dinv = lax.rsqrt(deg)[:, None]
    g1_o[...] = jnp.dot(x_r[...], w1_r[...],
                        preferred_element_type=jnp.float32) * dinv
    te_o[...] = jnp.maximum(
        jnp.dot(te_r[...], wt_r[...], preferred_element_type=jnp.float32)
        + bt_r[...], 0.0)
    dv_o[...] = dinv


def _tc_mid_temb_body(s_r, g_r, dv_r, b_r, w_r, te_r, gn_o):
    sr = s_r[...]
    dv = dv_r[...]
    h = jnp.maximum((sr[0] + sr[1] + g_r[...]) * dv + b_r[...], 0.0) + te_r[...]
    gn_o[...] = jnp.dot(h, w_r[...], preferred_element_type=jnp.float32) * dv


def _tc_mid_body(s_r, g_r, dv_r, b_r, w_r, gn_o):
    sr = s_r[...]
    dv = dv_r[...]
    h = jnp.maximum((sr[0] + sr[1] + g_r[...]) * dv + b_r[...], 0.0)
    gn_o[...] = jnp.dot(h, w_r[...], preferred_element_type=jnp.float32) * dv


def _tc_pool_body(s_r, g_r, dv_r, b_r, ba_r, ms_o, mc_o):
    i = pl.program_id(0)
    sr = s_r[...]
    h = jnp.maximum((sr[0] + sr[1] + g_r[...]) * dv_r[...] + b_r[...], 0.0)
    bb = ba_r[0]
    gids = lax.broadcasted_iota(jnp.int32, (_G, _BN), 0)
    mask = (gids == bb).astype(jnp.float32)
    ps = jnp.dot(mask, h, preferred_element_type=jnp.float32)
    pc = jnp.broadcast_to(jnp.sum(mask, axis=1, keepdims=True), (_G, _H))

    @pl.when(i == 0)
    def _():
        ms_o[...] = ps
        mc_o[...] = pc

    @pl.when(i != 0)
    def _():
        ms_o[...] = ms_o[...] + ps
        mc_o[...] = mc_o[...] + pc


def _tc_out_body(ms_r, mc_r, wo_r, bo_r, o_r):
    pooled = ms_r[...] / jnp.maximum(mc_r[...], 1.0)
    o_r[...] = jnp.dot(pooled, wo_r[...],
                       preferred_element_type=jnp.float32) + bo_r[...]


def _row_spec(i):
    return (i, 0)


def kernel(x, edge_index, t_embedding, batch, Wt, bt, W1, b1, W2, b2, W3, b3,
           Wo, bo):
    src = edge_index[0]
    dst = edge_index[1]
    zacc = jnp.zeros((_NPAD, _H), jnp.float32)
    onesr = jnp.ones((_CH, _H), jnp.float32)

    deg2 = _sc_deg(dst, onesr, zacc)

    grid = (_N // _BN,)
    row = pl.BlockSpec((_BN, _H), _row_spec)
    col1 = pl.BlockSpec((_BN, 1), _row_spec)
    wsp = pl.BlockSpec((_H, _H), lambda i: (0, 0))
    bsp = pl.BlockSpec((1, _H), lambda i: (0, 0))
    ssp = pl.BlockSpec((_NC, _BN, _H), lambda i: (0, i, 0))

    g1, temb, dinv = pl.pallas_call(
        _tc1_body,
        grid=grid,
        in_specs=[row, row, ssp, wsp, wsp, bsp],
        out_specs=[row, row, col1],
        out_shape=[jax.ShapeDtypeStruct((_N, _H), jnp.float32),
                   jax.ShapeDtypeStruct((_N, _H), jnp.float32),
                   jax.ShapeDtypeStruct((_N, 1), jnp.float32)],
    )(x, t_embedding, deg2, W1, Wt, bt.reshape(1, _H))

    s1 = _sc_edge(g1, src, dst, zacc)

    g2 = pl.pallas_call(
        _tc_mid_temb_body,
        grid=grid,
        in_specs=[ssp, row, col1, bsp, wsp, row],
        out_specs=row,
        out_shape=jax.ShapeDtypeStruct((_N, _H), jnp.float32),
    )(s1, g1, dinv, b1.reshape(1, _H), W2, temb)

    s2 = _sc_edge(g2, src, dst, zacc)

    g3 = pl.pallas_call(
        _tc_mid_body,
        grid=grid,
        in_specs=[ssp, row, col1, bsp, wsp],
        out_specs=row,
        out_shape=jax.ShapeDtypeStruct((_N, _H), jnp.float32),
    )(s2, g2, dinv, b2.reshape(1, _H), W3)

    s3 = _sc_edge(g3, src, dst, zacc)

    msum, mcnt = pl.pallas_call(
        _tc_pool_body,
        grid=grid,
        in_specs=[ssp, row, col1, bsp,
                  pl.BlockSpec((1, 1, _BN), lambda i: (i, 0, 0))],
        out_specs=[pl.BlockSpec((_G, _H), lambda i: (0, 0)),
                   pl.BlockSpec((_G, _H), lambda i: (0, 0))],
        out_shape=[jax.ShapeDtypeStruct((_G, _H), jnp.float32),
                   jax.ShapeDtypeStruct((_G, _H), jnp.float32)],
    )(s3, g3, dinv, b3.reshape(1, _H),
      batch.reshape(_N // _BN, 1, _BN))

    wo_pad = jnp.zeros((_H, _H), jnp.float32).at[:, :_NT].set(Wo)
    bo_pad = jnp.zeros((1, _H), jnp.float32).at[0, :_NT].set(bo)

    out = pl.pallas_call(
        _tc_out_body,
        grid=(1,),
        in_specs=[pl.BlockSpec((_G, _H), lambda i: (0, 0)),
                  pl.BlockSpec((_G, _H), lambda i: (0, 0)),
                  wsp, bsp],
        out_specs=pl.BlockSpec((_G, _H), lambda i: (0, 0)),
        out_shape=jax.ShapeDtypeStruct((_G, _H), jnp.float32),
    )(msum, mcnt, wo_pad, bo_pad)

    return out[:, :_NT]
